# baseline (speedup 1.0000x reference)
import functools

import jax
import jax.numpy as jnp
from jax import lax
from jax.experimental import pallas as pl
from jax.experimental.pallas import tpu as pltpu
from jax.experimental.pallas import tpu_sc as plsc

N_NODES = 10000
D_FEAT = 128
N_EDGES = 320000

NUM_CORES = 2
NUM_SUBCORES = 16
NUM_WORKERS = NUM_CORES * NUM_SUBCORES
EDGES_PER_WORKER = N_EDGES // NUM_WORKERS
CHUNK = 80
NUM_PASSES = 5
CHUNKS_PER_PASS = EDGES_PER_WORKER // (CHUNK * NUM_PASSES)
NBUF = 4
ROWS_PER_TILE = 624
TAIL_ROW0 = NUM_SUBCORES * ROWS_PER_TILE
TAIL_ROWS = N_NODES - TAIL_ROW0


def _sc_segment_sum(x, src, dst, zeros):
    mesh = plsc.VectorSubcoreMesh(core_axis_name="c", subcore_axis_name="s")

    @functools.partial(
        pl.kernel,
        mesh=mesh,
        out_type=jax.ShapeDtypeStruct((NUM_CORES, N_NODES, D_FEAT), jnp.float32),
        scratch_types=[
            pltpu.VMEM((CHUNKS_PER_PASS, CHUNK), jnp.int32),
            pltpu.VMEM((CHUNKS_PER_PASS, CHUNK), jnp.int32),
            pltpu.VMEM((CHUNK, D_FEAT), jnp.float32),
            pltpu.VMEM((CHUNK, D_FEAT), jnp.float32),
            pltpu.VMEM((CHUNK, D_FEAT), jnp.float32),
            pltpu.VMEM((CHUNK, D_FEAT), jnp.float32),
            pltpu.VMEM_SHARED((N_NODES, D_FEAT), jnp.float32),
            pltpu.SemaphoreType.DMA,
            pltpu.SemaphoreType.DMA,
            pltpu.SemaphoreType.DMA,
            pltpu.SemaphoreType.DMA,
            pltpu.SemaphoreType.DMA,
        ],
    )
    def sc_kernel(x_hbm, src_hbm, dst_hbm, zeros_hbm, out_hbm,
                  src_v, dst_v, rows_a, rows_b, rows_c, rows_d, acc,
                  sem_a, sem_b, sem_c, sem_d, zsem):
        cid = lax.axis_index("c")
        sid = lax.axis_index("s")
        wid = sid * NUM_CORES + cid

        row0 = sid * ROWS_PER_TILE
        pltpu.async_copy(zeros_hbm.at[pl.ds(row0, ROWS_PER_TILE)],
                         acc.at[pl.ds(row0, ROWS_PER_TILE)], zsem)

        @pl.when(sid == NUM_SUBCORES - 1)
        def _zero_tail():
            pltpu.async_copy(zeros_hbm.at[pl.ds(TAIL_ROW0, TAIL_ROWS)],
                             acc.at[pl.ds(TAIL_ROW0, TAIL_ROWS)], zsem)

        bufs = (rows_a, rows_b, rows_c, rows_d)
        sems = (sem_a, sem_b, sem_c, sem_d)

        for p in range(NUM_PASSES):
            pltpu.sync_copy(src_hbm.at[wid, p], src_v)
            pltpu.sync_copy(dst_hbm.at[wid, p], dst_v)

            if p == 0:
                pltpu.make_async_copy(
                    zeros_hbm.at[pl.ds(row0, ROWS_PER_TILE)],
                    acc.at[pl.ds(row0, ROWS_PER_TILE)], zsem).wait()

                @pl.when(sid == NUM_SUBCORES - 1)
                def _zero_tail_wait():
                    pltpu.make_async_copy(
                        zeros_hbm.at[pl.ds(TAIL_ROW0, TAIL_ROWS)],
                        acc.at[pl.ds(TAIL_ROW0, TAIL_ROWS)], zsem).wait()

                plsc.subcore_barrier()

            for b in range(NBUF):
                pltpu.async_copy(x_hbm.at[src_v.at[b]], bufs[b], sems[b])

            def body(g, carry):
                j = g * NBUF
                for b in range(NBUF):
                    buf, sem = bufs[b], sems[b]
                    pltpu.make_async_copy(x_hbm.at[src_v.at[j + b]], buf, sem).wait()
                    pltpu.sync_copy(buf, acc.at[dst_v.at[j + b]], add=True)

                    @pl.when(j + b + NBUF < CHUNKS_PER_PASS)
                    def _next():
                        pltpu.async_copy(x_hbm.at[src_v.at[j + b + NBUF]], buf, sem)

                return carry

            lax.fori_loop(0, CHUNKS_PER_PASS // NBUF, body, 0, unroll=False)

            for t in range(NBUF * (CHUNKS_PER_PASS // NBUF), CHUNKS_PER_PASS):
                b = t % NBUF
                pltpu.make_async_copy(x_hbm.at[src_v.at[t]], bufs[b], sems[b]).wait()
                pltpu.sync_copy(bufs[b], acc.at[dst_v.at[t]], add=True)

        plsc.subcore_barrier()
        pltpu.sync_copy(acc.at[pl.ds(row0, ROWS_PER_TILE)],
                        out_hbm.at[cid, pl.ds(row0, ROWS_PER_TILE)])

        @pl.when(sid == NUM_SUBCORES - 1)
        def _out_tail():
            pltpu.sync_copy(acc.at[pl.ds(TAIL_ROW0, TAIL_ROWS)],
                            out_hbm.at[cid, pl.ds(TAIL_ROW0, TAIL_ROWS)])

    return sc_kernel(x, src, dst, zeros)


def _mm_body(p_ref, w_ref, o_ref):
    t = p_ref[0] + p_ref[1]
    o_ref[...] = jnp.dot(t, w_ref[...], preferred_element_type=jnp.float32)


def _tc_add_matmul(partials, W):
    bm = 1000
    return pl.pallas_call(
        _mm_body,
        grid=(N_NODES // bm,),
        in_specs=[
            pl.BlockSpec((NUM_CORES, bm, D_FEAT), lambda i: (0, i, 0)),
            pl.BlockSpec((D_FEAT, D_FEAT), lambda i: (0, 0)),
        ],
        out_specs=pl.BlockSpec((bm, D_FEAT), lambda i: (i, 0)),
        out_shape=jax.ShapeDtypeStruct((N_NODES, D_FEAT), jnp.float32),
    )(partials, W)


def kernel(x, edge_index, use_hist, W):
    del use_hist
    shp = (NUM_WORKERS, NUM_PASSES, CHUNKS_PER_PASS, CHUNK)
    src = edge_index[0].astype(jnp.int32).reshape(shp)
    dst = edge_index[1].astype(jnp.int32).reshape(shp)
    zeros = jnp.zeros((N_NODES, D_FEAT), jnp.float32)
    partials = _sc_segment_sum(x, src, dst, zeros)
    return _tc_add_matmul(partials, W)


# speedup vs baseline: 1.0031x; 1.0031x over previous
import functools

import jax
import jax.numpy as jnp
from jax import lax
from jax.experimental import pallas as pl
from jax.experimental.pallas import tpu as pltpu
from jax.experimental.pallas import tpu_sc as plsc

N_NODES = 10000
D_FEAT = 128
N_EDGES = 320000

NUM_CORES = 2
NUM_SUBCORES = 16
NUM_WORKERS = NUM_CORES * NUM_SUBCORES
EDGES_PER_WORKER = N_EDGES // NUM_WORKERS
CHUNK = 100
NUM_PASSES = 4
CHUNKS_PER_PASS = EDGES_PER_WORKER // (CHUNK * NUM_PASSES)
NBUF = 3
ROWS_PER_TILE = 624
TAIL_ROW0 = NUM_SUBCORES * ROWS_PER_TILE
TAIL_ROWS = N_NODES - TAIL_ROW0


def _sc_segment_sum(x, src, dst, zeros):
    mesh = plsc.VectorSubcoreMesh(core_axis_name="c", subcore_axis_name="s")

    @functools.partial(
        pl.kernel,
        mesh=mesh,
        out_type=jax.ShapeDtypeStruct((NUM_CORES, N_NODES, D_FEAT), jnp.float32),
        scratch_types=[
            pltpu.VMEM((CHUNKS_PER_PASS, CHUNK), jnp.int32),
            pltpu.VMEM((CHUNKS_PER_PASS, CHUNK), jnp.int32),
            pltpu.VMEM((CHUNK, D_FEAT), jnp.float32),
            pltpu.VMEM((CHUNK, D_FEAT), jnp.float32),
            pltpu.VMEM((CHUNK, D_FEAT), jnp.float32),
            pltpu.VMEM_SHARED((N_NODES, D_FEAT), jnp.float32),
            pltpu.SemaphoreType.DMA,
            pltpu.SemaphoreType.DMA,
            pltpu.SemaphoreType.DMA,
        ],
    )
    def sc_kernel(x_hbm, src_hbm, dst_hbm, zeros_hbm, out_hbm,
                  src_v, dst_v, rows_a, rows_b, rows_c, acc,
                  sem_a, sem_b, sem_c):
        cid = lax.axis_index("c")
        sid = lax.axis_index("s")
        wid = sid * NUM_CORES + cid

        row0 = sid * ROWS_PER_TILE
        pltpu.sync_copy(zeros_hbm.at[pl.ds(row0, ROWS_PER_TILE)],
                        acc.at[pl.ds(row0, ROWS_PER_TILE)])

        @pl.when(sid == NUM_SUBCORES - 1)
        def _zero_tail():
            pltpu.sync_copy(zeros_hbm.at[pl.ds(TAIL_ROW0, TAIL_ROWS)],
                            acc.at[pl.ds(TAIL_ROW0, TAIL_ROWS)])

        plsc.subcore_barrier()

        bufs = (rows_a, rows_b, rows_c)
        sems = (sem_a, sem_b, sem_c)

        for p in range(NUM_PASSES):
            pltpu.sync_copy(src_hbm.at[wid, p], src_v)
            pltpu.sync_copy(dst_hbm.at[wid, p], dst_v)

            for b in range(NBUF):
                pltpu.async_copy(x_hbm.at[src_v.at[b]], bufs[b], sems[b])

            def body(g, carry):
                j = g * NBUF
                for b in range(NBUF):
                    buf, sem = bufs[b], sems[b]
                    pltpu.make_async_copy(x_hbm.at[src_v.at[j + b]], buf, sem).wait()
                    pltpu.sync_copy(buf, acc.at[dst_v.at[j + b]], add=True)

                    @pl.when(j + b + NBUF < CHUNKS_PER_PASS)
                    def _next():
                        pltpu.async_copy(x_hbm.at[src_v.at[j + b + NBUF]], buf, sem)

                return carry

            lax.fori_loop(0, CHUNKS_PER_PASS // NBUF, body, 0, unroll=False)

            for t in range(NBUF * (CHUNKS_PER_PASS // NBUF), CHUNKS_PER_PASS):
                b = t % NBUF
                pltpu.make_async_copy(x_hbm.at[src_v.at[t]], bufs[b], sems[b]).wait()
                pltpu.sync_copy(bufs[b], acc.at[dst_v.at[t]], add=True)

        plsc.subcore_barrier()
        pltpu.sync_copy(acc.at[pl.ds(row0, ROWS_PER_TILE)],
                        out_hbm.at[cid, pl.ds(row0, ROWS_PER_TILE)])

        @pl.when(sid == NUM_SUBCORES - 1)
        def _out_tail():
            pltpu.sync_copy(acc.at[pl.ds(TAIL_ROW0, TAIL_ROWS)],
                            out_hbm.at[cid, pl.ds(TAIL_ROW0, TAIL_ROWS)])

    return sc_kernel(x, src, dst, zeros)


def _mm_body(p_ref, w_ref, o_ref):
    t = p_ref[0] + p_ref[1]
    o_ref[...] = jnp.dot(t, w_ref[...], preferred_element_type=jnp.float32)


def _tc_add_matmul(partials, W):
    bm = 1000
    return pl.pallas_call(
        _mm_body,
        grid=(N_NODES // bm,),
        in_specs=[
            pl.BlockSpec((NUM_CORES, bm, D_FEAT), lambda i: (0, i, 0)),
            pl.BlockSpec((D_FEAT, D_FEAT), lambda i: (0, 0)),
        ],
        out_specs=pl.BlockSpec((bm, D_FEAT), lambda i: (i, 0)),
        out_shape=jax.ShapeDtypeStruct((N_NODES, D_FEAT), jnp.float32),
    )(partials, W)


def kernel(x, edge_index, use_hist, W):
    del use_hist
    shp = (NUM_WORKERS, NUM_PASSES, CHUNKS_PER_PASS, CHUNK)
    src = edge_index[0].astype(jnp.int32).reshape(shp)
    dst = edge_index[1].astype(jnp.int32).reshape(shp)
    zeros = jnp.zeros((N_NODES, D_FEAT), jnp.float32)
    partials = _sc_segment_sum(x, src, dst, zeros)
    return _tc_add_matmul(partials, W)
